# Initial kernel scaffold; baseline (speedup 1.0000x reference)
import jax
import jax.numpy as jnp
from jax.experimental import pallas as pl


def kernel(x, edge_index_0, edge_index_1, W_self, W_neigh, b):
    raise NotImplementedError("write your pallas kernel here")


# speedup vs baseline: 2.0977x; 2.0977x over previous
import functools

import jax
import jax.numpy as jnp
from jax import lax
from jax.experimental import pallas as pl
from jax.experimental.pallas import tpu as pltpu
from jax.experimental.pallas import tpu_sc as plsc

N = 50000
E = 400000
D = 128
L = 3
ET = 2

NS = 16
CW = 32
NK = D // CW
B = 200
EPT = E // NS
NB = EPT // B
NP = 50176
RPT = NP // NS
DR = 448
ND = RPT // DR

_MESH = dict(
    mesh=plsc.VectorSubcoreMesh(core_axis_name="c", subcore_axis_name="s"),
    compiler_params=pltpu.CompilerParams(use_tc_tiling_on_sc=False),
)

_mk_out = [jax.ShapeDtypeStruct((ET, NP, CW), jnp.float32) for _ in range(NK)]


@functools.partial(
    pl.kernel,
    out_type=_mk_out,
    scratch_types=[
        pltpu.VMEM_SHARED((NP, CW), jnp.float32),
        pltpu.VMEM((B,), jnp.int32),
        pltpu.VMEM((B,), jnp.int32),
        pltpu.VMEM((B, CW), jnp.float32),
        pltpu.VMEM((DR, CW), jnp.float32),
        pltpu.SemaphoreType.DMA,
    ],
    **_MESH,
)
def _msum(h0, h1, h2, h3, srcs, dsts, zz, m0, m1, m2, m3,
          acc, src_v, dst_v, rows_v, zbuf, gsem):
    hs = (h0, h1, h2, h3)
    outs = (m0, m1, m2, m3)
    c = lax.axis_index("c")
    s = lax.axis_index("s")
    slab0 = s * RPT
    ebase = c * E + s * EPT
    for k in range(NK):
        pltpu.sync_copy(zz, zbuf)
        for j in range(ND):
            pltpu.sync_copy(zbuf, acc.at[pl.ds(slab0 + j * DR, DR)])
        plsc.subcore_barrier()

        def edge_batch(bi, carry):
            off = ebase + bi * B
            pltpu.sync_copy(srcs.at[pl.ds(off, B)], src_v)
            pltpu.sync_copy(dsts.at[pl.ds(off, B)], dst_v)
            pltpu.async_copy(hs[k].at[src_v], rows_v, gsem).wait()
            pltpu.sync_copy(rows_v, acc.at[dst_v], add=True)
            return carry

        lax.fori_loop(0, NB, edge_batch, 0)
        plsc.subcore_barrier()
        for j in range(ND):
            r0 = slab0 + j * DR
            pltpu.sync_copy(acc.at[pl.ds(r0, DR)], zbuf)
            pltpu.sync_copy(zbuf, outs[k].at[c, pl.ds(r0, DR)])
        plsc.subcore_barrier()


@functools.partial(
    pl.kernel,
    out_type=jax.ShapeDtypeStruct((ET, NP, 8), jnp.float32),
    scratch_types=[
        pltpu.VMEM_SHARED((NP, 8), jnp.float32),
        pltpu.VMEM((B,), jnp.int32),
        pltpu.VMEM((B, 8), jnp.float32),
        pltpu.VMEM((DR, 8), jnp.float32),
    ],
    **_MESH,
)
def _deg(dsts, ones8, z8, out, accd, dst_v, ones_v, zbuf8):
    c = lax.axis_index("c")
    s = lax.axis_index("s")
    slab0 = s * RPT
    ebase = c * E + s * EPT
    pltpu.sync_copy(ones8, ones_v)
    pltpu.sync_copy(z8, zbuf8)
    for j in range(ND):
        pltpu.sync_copy(zbuf8, accd.at[pl.ds(slab0 + j * DR, DR)])
    plsc.subcore_barrier()

    def edge_batch(bi, carry):
        off = ebase + bi * B
        pltpu.sync_copy(dsts.at[pl.ds(off, B)], dst_v)
        pltpu.sync_copy(ones_v, accd.at[dst_v], add=True)
        return carry

    lax.fori_loop(0, NB, edge_batch, 0)
    plsc.subcore_barrier()
    for j in range(ND):
        r0 = slab0 + j * DR
        pltpu.sync_copy(accd.at[pl.ds(r0, DR)], zbuf8)
        pltpu.sync_copy(zbuf8, out.at[c, pl.ds(r0, DR)])


BN = 1000


def _dense_body(refs, *, final):
    h_refs = refs[0:NK]
    m_refs = refs[NK:2 * NK]
    deg_ref, ws_ref, wn_ref, b_ref = refs[2 * NK:2 * NK + 4]
    out_refs = refs[2 * NK + 4:]
    h = jnp.concatenate([r[...] for r in h_refs], axis=1)
    out = jnp.zeros((BN, D), jnp.float32)
    for e in range(ET):
        inv = 1.0 / jnp.maximum(deg_ref[e][:, 0:1], 1.0)
        me = jnp.concatenate([r[e] for r in m_refs], axis=1)
        he = (
            jnp.dot(h, ws_ref[e], preferred_element_type=jnp.float32)
            + jnp.dot(me * inv, wn_ref[e], preferred_element_type=jnp.float32)
            + b_ref[e]
        )
        if not final:
            he = jnp.maximum(he, 0.0)
        out = out + he
    if final:
        out_refs[0][...] = out
    else:
        for k in range(NK):
            out_refs[k][...] = out[:, k * CW:(k + 1) * CW]


def _dense(hs, ms, deg, ws, wn, bb, *, final):
    if final:
        out_shape = jax.ShapeDtypeStruct((N, D), jnp.float32)
        out_specs = pl.BlockSpec((BN, D), lambda i: (i, 0))
    else:
        out_shape = [jax.ShapeDtypeStruct((N, CW), jnp.float32) for _ in range(NK)]
        out_specs = [pl.BlockSpec((BN, CW), lambda i: (i, 0)) for _ in range(NK)]
    body = lambda *refs: _dense_body(refs, final=final)
    return pl.pallas_call(
        body,
        grid=(N // BN,),
        in_specs=(
            [pl.BlockSpec((BN, CW), lambda i: (i, 0)) for _ in range(NK)]
            + [pl.BlockSpec((ET, BN, CW), lambda i: (0, i, 0)) for _ in range(NK)]
            + [
                pl.BlockSpec((ET, BN, 8), lambda i: (0, i, 0)),
                pl.BlockSpec((ET, D, D), lambda i: (0, 0, 0)),
                pl.BlockSpec((ET, D, D), lambda i: (0, 0, 0)),
                pl.BlockSpec((ET, D), lambda i: (0, 0)),
            ]
        ),
        out_specs=out_specs,
        out_shape=out_shape,
    )(*hs, *ms, deg, ws, wn, bb)


def kernel(x, edge_index_0, edge_index_1, W_self, W_neigh, b):
    srcs = jnp.concatenate([edge_index_0[0], edge_index_1[0]])
    dsts = jnp.concatenate([edge_index_0[1], edge_index_1[1]])
    zz = jnp.zeros((DR, CW), jnp.float32)
    ones8 = jnp.ones((B, 8), jnp.float32)
    z8 = jnp.zeros((DR, 8), jnp.float32)
    deg = _deg(dsts, ones8, z8)
    hs = [x[:, k * CW:(k + 1) * CW] for k in range(NK)]
    for l in range(L):
        ms = _msum(*hs, srcs, dsts, zz)
        out = _dense(hs, ms, deg, W_self[l], W_neigh[l], b[l], final=(l == L - 1))
        hs = out
    return out


# speedup vs baseline: 3.1798x; 1.5159x over previous
import functools

import jax
import jax.numpy as jnp
from jax import lax
from jax.experimental import pallas as pl
from jax.experimental.pallas import tpu as pltpu
from jax.experimental.pallas import tpu_sc as plsc

N = 50000
E = 400000
D = 128
L = 3
ET = 2

NS = 16
CW = 32
NK = D // CW
B = 200
EPT = E // NS
NB = EPT // B
CPB = 5
NCH = NB // CPB
NBR = 2 * E // B
NP = 50176
RPT = NP // NS
DR = 448
ND = RPT // DR

_MESH = dict(
    mesh=plsc.VectorSubcoreMesh(core_axis_name="c", subcore_axis_name="s"),
    compiler_params=pltpu.CompilerParams(use_tc_tiling_on_sc=False),
)

_mk_out = [jax.ShapeDtypeStruct((ET, NP, CW), jnp.float32) for _ in range(NK)]


@functools.partial(
    pl.kernel,
    out_type=_mk_out,
    scratch_types=[
        pltpu.VMEM_SHARED((NP, CW), jnp.float32),
        pltpu.VMEM((2, CPB, B), jnp.int32),
        pltpu.VMEM((2, CPB, B), jnp.int32),
        pltpu.VMEM((2, B, CW), jnp.float32),
        pltpu.SemaphoreType.DMA,
        pltpu.SemaphoreType.DMA,
        pltpu.SemaphoreType.DMA,
        pltpu.SemaphoreType.DMA,
    ],
    **_MESH,
)
def _msum(h0, h1, h2, h3, srcs, dsts, zz, m0, m1, m2, m3,
          acc, src_b, dst_b, rows2, gsem, ssem, isem, zsem):
    hs = (h0, h1, h2, h3)
    outs = (m0, m1, m2, m3)
    c = lax.axis_index("c")
    s = lax.axis_index("s")
    slab0 = s * RPT
    gb0 = c * (E // B) + s * NB
    for k in range(NK):
        for j in range(ND):
            pltpu.async_copy(zz, acc.at[pl.ds(slab0 + j * DR, DR)], zsem)
        for j in range(ND):
            pltpu.make_async_copy(zz, acc.at[pl.ds(slab0 + j * DR, DR)], zsem).wait()
        plsc.subcore_barrier()

        pltpu.sync_copy(srcs.at[pl.ds(gb0, CPB)], src_b.at[0])
        pltpu.sync_copy(dsts.at[pl.ds(gb0, CPB)], dst_b.at[0])
        pltpu.async_copy(hs[k].at[src_b.at[0, 0]], rows2.at[0], gsem)

        def body(j, carry):
            p = j % 2
            q = (j // CPB) % 2
            r = j % CPB
            pltpu.make_async_copy(hs[k].at[src_b.at[q, r]], rows2.at[p], gsem).wait()
            @pl.when(j >= 1)
            def _():
                pltpu.make_async_copy(
                    rows2.at[1 - p], acc.at[dst_b.at[q, r]], ssem).wait()
            ch1 = j // CPB + 1
            @pl.when((r == 0) & (ch1 < NCH))
            def _():
                row = gb0 + ch1 * CPB
                pltpu.async_copy(srcs.at[pl.ds(row, CPB)], src_b.at[1 - q], isem)
                pltpu.async_copy(dsts.at[pl.ds(row, CPB)], dst_b.at[1 - q], isem)
            @pl.when(j + 1 < NB)
            def _():
                j1 = j + 1
                q1 = (j1 // CPB) % 2
                r1 = j1 % CPB
                @pl.when(r1 == 0)
                def _():
                    pltpu.make_async_copy(
                        srcs.at[pl.ds(gb0, CPB)], src_b.at[q1], isem).wait()
                    pltpu.make_async_copy(
                        dsts.at[pl.ds(gb0, CPB)], dst_b.at[q1], isem).wait()
                pltpu.async_copy(hs[k].at[src_b.at[q1, r1]], rows2.at[1 - p], gsem)
            pltpu.async_copy(rows2.at[p], acc.at[dst_b.at[q, r]], ssem, add=True)
            return carry

        lax.fori_loop(0, NB, body, 0)
        pltpu.make_async_copy(
            rows2.at[(NB - 1) % 2], acc.at[dst_b.at[0, 0]], ssem).wait()
        plsc.subcore_barrier()
        for j in range(ND):
            r0 = slab0 + j * DR
            pltpu.async_copy(acc.at[pl.ds(r0, DR)], outs[k].at[c, pl.ds(r0, DR)], zsem)
        for j in range(ND):
            r0 = slab0 + j * DR
            pltpu.make_async_copy(
                acc.at[pl.ds(r0, DR)], outs[k].at[c, pl.ds(r0, DR)], zsem).wait()
        plsc.subcore_barrier()


@functools.partial(
    pl.kernel,
    out_type=jax.ShapeDtypeStruct((ET, NP, 8), jnp.float32),
    scratch_types=[
        pltpu.VMEM_SHARED((NP, 8), jnp.float32),
        pltpu.VMEM((B,), jnp.int32),
        pltpu.VMEM((B, 8), jnp.float32),
        pltpu.SemaphoreType.DMA,
    ],
    **_MESH,
)
def _deg(dsts, ones8, z8, out, accd, dst_v, ones_v, zsem):
    c = lax.axis_index("c")
    s = lax.axis_index("s")
    slab0 = s * RPT
    gb0 = c * (E // B) + s * NB
    pltpu.sync_copy(ones8, ones_v)
    for j in range(ND):
        pltpu.async_copy(z8, accd.at[pl.ds(slab0 + j * DR, DR)], zsem)
    for j in range(ND):
        pltpu.make_async_copy(z8, accd.at[pl.ds(slab0 + j * DR, DR)], zsem).wait()
    plsc.subcore_barrier()

    def edge_batch(bi, carry):
        pltpu.sync_copy(dsts.at[gb0 + bi], dst_v)
        pltpu.sync_copy(ones_v, accd.at[dst_v], add=True)
        return carry

    lax.fori_loop(0, NB, edge_batch, 0)
    plsc.subcore_barrier()
    for j in range(ND):
        r0 = slab0 + j * DR
        pltpu.async_copy(accd.at[pl.ds(r0, DR)], out.at[c, pl.ds(r0, DR)], zsem)
    for j in range(ND):
        r0 = slab0 + j * DR
        pltpu.make_async_copy(
            accd.at[pl.ds(r0, DR)], out.at[c, pl.ds(r0, DR)], zsem).wait()


BN = 1000


def _dense_body(refs, *, final):
    h_refs = refs[0:NK]
    m_refs = refs[NK:2 * NK]
    deg_ref, ws_ref, wn_ref, b_ref = refs[2 * NK:2 * NK + 4]
    out_refs = refs[2 * NK + 4:]
    h = jnp.concatenate([r[...] for r in h_refs], axis=1)
    out = jnp.zeros((BN, D), jnp.float32)
    for e in range(ET):
        inv = 1.0 / jnp.maximum(deg_ref[e][:, 0:1], 1.0)
        me = jnp.concatenate([r[e] for r in m_refs], axis=1)
        he = (
            jnp.dot(h, ws_ref[e], preferred_element_type=jnp.float32)
            + jnp.dot(me * inv, wn_ref[e], preferred_element_type=jnp.float32)
            + b_ref[e]
        )
        if not final:
            he = jnp.maximum(he, 0.0)
        out = out + he
    if final:
        out_refs[0][...] = out
    else:
        for k in range(NK):
            out_refs[k][...] = out[:, k * CW:(k + 1) * CW]


def _dense(hs, ms, deg, ws, wn, bb, *, final):
    if final:
        out_shape = jax.ShapeDtypeStruct((N, D), jnp.float32)
        out_specs = pl.BlockSpec((BN, D), lambda i: (i, 0))
    else:
        out_shape = [jax.ShapeDtypeStruct((N, CW), jnp.float32) for _ in range(NK)]
        out_specs = [pl.BlockSpec((BN, CW), lambda i: (i, 0)) for _ in range(NK)]
    body = lambda *refs: _dense_body(refs, final=final)
    return pl.pallas_call(
        body,
        grid=(N // BN,),
        in_specs=(
            [pl.BlockSpec((BN, CW), lambda i: (i, 0)) for _ in range(NK)]
            + [pl.BlockSpec((ET, BN, CW), lambda i: (0, i, 0)) for _ in range(NK)]
            + [
                pl.BlockSpec((ET, BN, 8), lambda i: (0, i, 0)),
                pl.BlockSpec((ET, D, D), lambda i: (0, 0, 0)),
                pl.BlockSpec((ET, D, D), lambda i: (0, 0, 0)),
                pl.BlockSpec((ET, D), lambda i: (0, 0)),
            ]
        ),
        out_specs=out_specs,
        out_shape=out_shape,
    )(*hs, *ms, deg, ws, wn, bb)


def kernel(x, edge_index_0, edge_index_1, W_self, W_neigh, b):
    srcs = jnp.concatenate([edge_index_0[0], edge_index_1[0]]).reshape(NBR, B)
    dsts = jnp.concatenate([edge_index_0[1], edge_index_1[1]]).reshape(NBR, B)
    zz = jnp.zeros((DR, CW), jnp.float32)
    ones8 = jnp.ones((B, 8), jnp.float32)
    z8 = jnp.zeros((DR, 8), jnp.float32)
    deg = _deg(dsts, ones8, z8)
    hs = [x[:, k * CW:(k + 1) * CW] for k in range(NK)]
    for l in range(L):
        ms = _msum(*hs, srcs, dsts, zz)
        out = _dense(hs, ms, deg, W_self[l], W_neigh[l], b[l], final=(l == L - 1))
        hs = out
    return out


# speedup vs baseline: 3.7409x; 1.1764x over previous
import functools

import jax
import jax.numpy as jnp
from jax import lax
from jax.experimental import pallas as pl
from jax.experimental.pallas import tpu as pltpu
from jax.experimental.pallas import tpu_sc as plsc

N = 50000
E = 400000
D = 128
L = 3
ET = 2

NS = 16
CW = 32
NK = D // CW
B = 200
EPT = E // NS
NB = EPT // B
CPB = 5
NCH = NB // CPB
NBR = 2 * E // B
NP = 50176
RPT = NP // NS
DR = 448
ND = RPT // DR

_MESH = dict(
    mesh=plsc.VectorSubcoreMesh(core_axis_name="c", subcore_axis_name="s"),
    compiler_params=pltpu.CompilerParams(use_tc_tiling_on_sc=False),
)

_mk_out = [jax.ShapeDtypeStruct((ET, NP, CW), jnp.float32) for _ in range(NK)]


def _make_msum(with_deg):
    out_type = ([jax.ShapeDtypeStruct((ET, NP, CW), jnp.float32)] if with_deg else []) + _mk_out

    def body(h0, h1, h2, h3, epk, zz, oo, *rest):
        n_out = NK + (1 if with_deg else 0)
        outs = rest[:n_out]
        acc, idx2, rows4, gsem, ssem, isem, zsem = rest[n_out:]
        hs = (h0, h1, h2, h3)
        c = lax.axis_index("c")
        s = lax.axis_index("s")
        slab0 = s * RPT
        gb0 = c * (E // B) + s * NB

        def zero_acc():
            for j in range(ND):
                pltpu.async_copy(zz, acc.at[pl.ds(slab0 + j * DR, DR)], zsem)
            for j in range(ND):
                pltpu.make_async_copy(zz, acc.at[pl.ds(slab0 + j * DR, DR)], zsem).wait()

        def drain(out_ref):
            for j in range(ND):
                r0 = slab0 + j * DR
                pltpu.async_copy(acc.at[pl.ds(r0, DR)], out_ref.at[c, pl.ds(r0, DR)], zsem)
            for j in range(ND):
                r0 = slab0 + j * DR
                pltpu.make_async_copy(
                    acc.at[pl.ds(r0, DR)], out_ref.at[c, pl.ds(r0, DR)], zsem).wait()

        def prefetch_and_scatter_loop(gather_table):
            def body_fn(j, carry):
                p = j % 4
                sp = j % 2
                q = (j // CPB) % 2
                r = j % CPB
                if gather_table is not None:
                    pltpu.make_async_copy(
                        gather_table.at[idx2.at[q, r, 0]], rows4.at[p],
                        gsem.at[sp]).wait()
                src_slot = rows4.at[p] if gather_table is not None else rows4.at[0]

                @pl.when(j >= 2)
                def _():
                    pltpu.make_async_copy(
                        rows4.at[0] if gather_table is None else rows4.at[(j + 2) % 4],
                        acc.at[idx2.at[q, r, 1]], ssem.at[sp]).wait()

                ch1 = j // CPB + 1
                @pl.when((r == 2) & (ch1 < NCH))
                def _():
                    pltpu.async_copy(
                        epk.at[pl.ds(gb0 + ch1 * CPB, CPB)], idx2.at[1 - q], isem)

                if gather_table is not None:
                    @pl.when(j + 2 < NB)
                    def _():
                        j2 = j + 2
                        q2 = (j2 // CPB) % 2
                        r2 = j2 % CPB
                        @pl.when(r2 == 0)
                        def _():
                            pltpu.make_async_copy(
                                epk.at[pl.ds(gb0, CPB)], idx2.at[q2], isem).wait()
                        pltpu.async_copy(
                            gather_table.at[idx2.at[q2, r2, 0]], rows4.at[j2 % 4],
                            gsem.at[sp])
                else:
                    @pl.when((r == CPB - 1) & (j + 1 < NB))
                    def _():
                        q1 = ((j + 1) // CPB) % 2
                        pltpu.make_async_copy(
                            epk.at[pl.ds(gb0, CPB)], idx2.at[q1], isem).wait()

                pltpu.async_copy(src_slot, acc.at[idx2.at[q, r, 1]], ssem.at[sp],
                                 add=True)
                return carry

            pltpu.sync_copy(epk.at[pl.ds(gb0, CPB)], idx2.at[0])
            if gather_table is not None:
                pltpu.async_copy(gather_table.at[idx2.at[0, 0, 0]], rows4.at[0],
                                 gsem.at[0])
                pltpu.async_copy(gather_table.at[idx2.at[0, 1, 0]], rows4.at[1],
                                 gsem.at[1])
            lax.fori_loop(0, NB, body_fn, 0)
            pltpu.make_async_copy(
                rows4.at[(NB - 2) % 4] if gather_table is not None else rows4.at[0],
                acc.at[idx2.at[0, 0, 1]], ssem.at[(NB - 2) % 2]).wait()
            pltpu.make_async_copy(
                rows4.at[(NB - 1) % 4] if gather_table is not None else rows4.at[0],
                acc.at[idx2.at[0, 0, 1]], ssem.at[(NB - 1) % 2]).wait()

        phases = ([None] if with_deg else []) + list(range(NK))
        for i, ph in enumerate(phases):
            zero_acc()
            if ph is None:
                pltpu.sync_copy(oo, rows4.at[0])
            plsc.subcore_barrier()
            prefetch_and_scatter_loop(None if ph is None else hs[ph])
            plsc.subcore_barrier()
            drain(outs[i])
            plsc.subcore_barrier()

    return pl.kernel(
        body,
        out_type=out_type,
        scratch_types=[
            pltpu.VMEM_SHARED((NP, CW), jnp.float32),
            pltpu.VMEM((2, CPB, 2, B), jnp.int32),
            pltpu.VMEM((4, B, CW), jnp.float32),
            pltpu.SemaphoreType.DMA((2,)),
            pltpu.SemaphoreType.DMA((2,)),
            pltpu.SemaphoreType.DMA,
            pltpu.SemaphoreType.DMA,
        ],
        **_MESH,
    )


_msum0 = _make_msum(with_deg=True)
_msum = _make_msum(with_deg=False)


BN = 1000


def _dense_body(refs, *, final):
    h_refs = refs[0:NK]
    m_refs = refs[NK:2 * NK]
    deg_ref, ws_ref, wn_ref, b_ref = refs[2 * NK:2 * NK + 4]
    out_refs = refs[2 * NK + 4:]
    h = jnp.concatenate([r[...] for r in h_refs], axis=1)
    out = jnp.zeros((BN, D), jnp.float32)
    for e in range(ET):
        inv = 1.0 / jnp.maximum(deg_ref[e][:, 0:1], 1.0)
        me = jnp.concatenate([r[e] for r in m_refs], axis=1)
        he = (
            jnp.dot(h, ws_ref[e], preferred_element_type=jnp.float32)
            + jnp.dot(me * inv, wn_ref[e], preferred_element_type=jnp.float32)
            + b_ref[e]
        )
        if not final:
            he = jnp.maximum(he, 0.0)
        out = out + he
    if final:
        out_refs[0][...] = out
    else:
        for k in range(NK):
            out_refs[k][...] = out[:, k * CW:(k + 1) * CW]


def _dense(hs, ms, deg, ws, wn, bb, *, final):
    if final:
        out_shape = jax.ShapeDtypeStruct((N, D), jnp.float32)
        out_specs = pl.BlockSpec((BN, D), lambda i: (i, 0))
    else:
        out_shape = [jax.ShapeDtypeStruct((N, CW), jnp.float32) for _ in range(NK)]
        out_specs = [pl.BlockSpec((BN, CW), lambda i: (i, 0)) for _ in range(NK)]
    body = lambda *refs: _dense_body(refs, final=final)
    return pl.pallas_call(
        body,
        grid=(N // BN,),
        in_specs=(
            [pl.BlockSpec((BN, CW), lambda i: (i, 0)) for _ in range(NK)]
            + [pl.BlockSpec((ET, BN, CW), lambda i: (0, i, 0)) for _ in range(NK)]
            + [
                pl.BlockSpec((ET, BN, CW), lambda i: (0, i, 0)),
                pl.BlockSpec((ET, D, D), lambda i: (0, 0, 0)),
                pl.BlockSpec((ET, D, D), lambda i: (0, 0, 0)),
                pl.BlockSpec((ET, D), lambda i: (0, 0)),
            ]
        ),
        out_specs=out_specs,
        out_shape=out_shape,
    )(*hs, *ms, deg, ws, wn, bb)


def kernel(x, edge_index_0, edge_index_1, W_self, W_neigh, b):
    se = jnp.concatenate([edge_index_0[0], edge_index_1[0]]).reshape(NBR, B)
    de = jnp.concatenate([edge_index_0[1], edge_index_1[1]]).reshape(NBR, B)
    epk = jnp.stack([se, de], axis=1)
    zz = jnp.zeros((DR, CW), jnp.float32)
    oo = jnp.ones((B, CW), jnp.float32)
    hs = [x[:, k * CW:(k + 1) * CW] for k in range(NK)]
    deg = None
    for l in range(L):
        if l == 0:
            deg, *ms = _msum0(*hs, epk, zz, oo)
        else:
            ms = _msum(*hs, epk, zz, oo)
        out = _dense(hs, ms, deg, W_self[l], W_neigh[l], b[l], final=(l == L - 1))
        hs = out
    return out


# speedup vs baseline: 5.0477x; 1.3493x over previous
import functools

import jax
import jax.numpy as jnp
from jax import lax
from jax.experimental import pallas as pl
from jax.experimental.pallas import tpu as pltpu
from jax.experimental.pallas import tpu_sc as plsc

N = 50000
E = 400000
D = 128
L = 3
ET = 2

NS = 16
CW = 32
NK = D // CW
B = 200
EPT = E // NS
NB = EPT // B
CPB = 5
NCH = NB // CPB
NBR = 2 * E // B
NP = 50176
RPT = NP // NS
DR = 448
ND = RPT // DR

_MESH = dict(
    mesh=plsc.VectorSubcoreMesh(core_axis_name="c", subcore_axis_name="s"),
    compiler_params=pltpu.CompilerParams(use_tc_tiling_on_sc=False),
)


def _make_msum(with_deg):
    out_type = [jax.ShapeDtypeStruct((ET, NP, D), jnp.float32)]
    if with_deg:
        out_type = [jax.ShapeDtypeStruct((ET, NP, D), jnp.float32)] + out_type

    def body(hv, epk, zz, oo, *rest):
        if with_deg:
            dg, mo = rest[0], rest[1]
            scr = rest[2:]
        else:
            mo = rest[0]
            scr = rest[1:]
        acc, idx2, rows4, gsem, ssem, isem, zsem = scr
        c = lax.axis_index("c")
        s = lax.axis_index("s")
        slab0 = s * RPT
        gb0 = c * (E // B) + s * NB

        def zero_acc():
            for j in range(ND):
                pltpu.async_copy(zz, acc.at[pl.ds(slab0 + j * DR, DR)], zsem)
            for j in range(ND):
                pltpu.make_async_copy(zz, acc.at[pl.ds(slab0 + j * DR, DR)], zsem).wait()

        def drain(out_ref, k):
            col = pl.ds(k * CW, CW)
            for j in range(ND):
                r0 = slab0 + j * DR
                pltpu.async_copy(acc.at[pl.ds(r0, DR)],
                                 out_ref.at[c, pl.ds(r0, DR), col], zsem)
            for j in range(ND):
                r0 = slab0 + j * DR
                pltpu.make_async_copy(acc.at[pl.ds(r0, DR)],
                                      out_ref.at[c, pl.ds(r0, DR), col], zsem).wait()

        def load_chunk(ch, slot, k, sem_copies):
            row = pl.ds(gb0 + ch * CPB, CPB)
            if k is not None:
                sem_copies.append((epk.at[row, k], idx2.at[slot, pl.ds(0, CPB), 0]))
            sem_copies.append((epk.at[row, NK], idx2.at[slot, pl.ds(0, CPB), 1]))

        def edge_loop(k):
            gather = k is not None
            nidx = 2 if gather else 1

            def chunk_load_async(ch, slot):
                cps = []
                load_chunk(ch, slot, k, cps)
                for src, dst in cps:
                    pltpu.async_copy(src, dst, isem)

            def chunk_load_wait():
                for _ in range(nidx):
                    pltpu.make_async_copy(
                        epk.at[pl.ds(gb0, CPB), NK],
                        idx2.at[0, pl.ds(0, CPB), 1], isem).wait()

            def body_fn(j, carry):
                p = j % 4
                sp = j % 2
                q = (j // CPB) % 2
                r = j % CPB
                if gather:
                    pltpu.make_async_copy(
                        hv.at[idx2.at[q, r, 0]], rows4.at[p], gsem.at[sp]).wait()
                src_slot = rows4.at[p] if gather else rows4.at[0]

                @pl.when(j >= 2)
                def _():
                    pltpu.make_async_copy(
                        rows4.at[(j + 2) % 4] if gather else rows4.at[0],
                        acc.at[idx2.at[q, r, 1]], ssem.at[sp]).wait()

                ch1 = j // CPB + 1
                @pl.when((r == 2) & (ch1 < NCH))
                def _():
                    chunk_load_async(ch1, 1 - q)

                if gather:
                    @pl.when(j + 2 < NB)
                    def _():
                        j2 = j + 2
                        q2 = (j2 // CPB) % 2
                        r2 = j2 % CPB
                        @pl.when(r2 == 0)
                        def _():
                            chunk_load_wait()
                        pltpu.async_copy(
                            hv.at[idx2.at[q2, r2, 0]], rows4.at[j2 % 4],
                            gsem.at[sp])
                else:
                    @pl.when((r == CPB - 1) & (j + 1 < NB))
                    def _():
                        chunk_load_wait()

                pltpu.async_copy(src_slot, acc.at[idx2.at[q, r, 1]], ssem.at[sp],
                                 add=True)
                return carry

            chunk_load_async(0, 0)
            chunk_load_wait()
            if gather:
                pltpu.async_copy(hv.at[idx2.at[0, 0, 0]], rows4.at[0], gsem.at[0])
                pltpu.async_copy(hv.at[idx2.at[0, 1, 0]], rows4.at[1], gsem.at[1])
            lax.fori_loop(0, NB, body_fn, 0)
            pltpu.make_async_copy(
                rows4.at[(NB - 2) % 4] if gather else rows4.at[0],
                acc.at[idx2.at[0, 0, 1]], ssem.at[(NB - 2) % 2]).wait()
            pltpu.make_async_copy(
                rows4.at[(NB - 1) % 4] if gather else rows4.at[0],
                acc.at[idx2.at[0, 0, 1]], ssem.at[(NB - 1) % 2]).wait()

        if with_deg:
            zero_acc()
            pltpu.sync_copy(oo, rows4.at[0])
            plsc.subcore_barrier()
            edge_loop(None)
            plsc.subcore_barrier()
            drain(dg, 0)
            plsc.subcore_barrier()
        for k in range(NK):
            zero_acc()
            plsc.subcore_barrier()
            edge_loop(k)
            plsc.subcore_barrier()
            drain(mo, k)
            plsc.subcore_barrier()

    return pl.kernel(
        body,
        out_type=out_type,
        scratch_types=[
            pltpu.VMEM_SHARED((NP, CW), jnp.float32),
            pltpu.VMEM((2, CPB, 2, B), jnp.int32),
            pltpu.VMEM((4, B, CW), jnp.float32),
            pltpu.SemaphoreType.DMA((2,)),
            pltpu.SemaphoreType.DMA((2,)),
            pltpu.SemaphoreType.DMA,
            pltpu.SemaphoreType.DMA,
        ],
        **_MESH,
    )


_msum0 = _make_msum(with_deg=True)
_msum = _make_msum(with_deg=False)


BN = 1000


def _dense_body(h_ref, m_ref, deg_ref, ws_ref, wn_ref, b_ref, out_ref, *, final):
    h = h_ref[...]
    out = jnp.zeros((BN, D), jnp.float32)
    for e in range(ET):
        inv = 1.0 / jnp.maximum(deg_ref[e][:, 0:1], 1.0)
        he = (
            jnp.dot(h, ws_ref[e], preferred_element_type=jnp.float32)
            + jnp.dot(m_ref[e] * inv, wn_ref[e], preferred_element_type=jnp.float32)
            + b_ref[e]
        )
        if not final:
            he = jnp.maximum(he, 0.0)
        out = out + he
    out_ref[...] = out


def _dense(h, ms, deg, ws, wn, bb, *, final):
    return pl.pallas_call(
        functools.partial(_dense_body, final=final),
        grid=(N // BN,),
        in_specs=[
            pl.BlockSpec((BN, D), lambda i: (i, 0)),
            pl.BlockSpec((ET, BN, D), lambda i: (0, i, 0)),
            pl.BlockSpec((ET, BN, D), lambda i: (0, i, 0)),
            pl.BlockSpec((ET, D, D), lambda i: (0, 0, 0)),
            pl.BlockSpec((ET, D, D), lambda i: (0, 0, 0)),
            pl.BlockSpec((ET, D), lambda i: (0, 0)),
        ],
        out_specs=pl.BlockSpec((BN, D), lambda i: (i, 0)),
        out_shape=jax.ShapeDtypeStruct((N, D), jnp.float32),
    )(h, ms, deg, ws, wn, bb)


def kernel(x, edge_index_0, edge_index_1, W_self, W_neigh, b):
    se = jnp.concatenate([edge_index_0[0], edge_index_1[0]]).reshape(NBR, B)
    de = jnp.concatenate([edge_index_0[1], edge_index_1[1]]).reshape(NBR, B)
    epk = jnp.stack([se * NK + k for k in range(NK)] + [de], axis=1)
    zz = jnp.zeros((DR, CW), jnp.float32)
    oo = jnp.ones((B, CW), jnp.float32)
    h = x
    deg = None
    for l in range(L):
        hv = h.reshape(N * NK, CW)
        if l == 0:
            deg, ms = _msum0(hv, epk, zz, oo)
        else:
            (ms,) = _msum(hv, epk, zz, oo)
        h = _dense(h, ms, deg, W_self[l], W_neigh[l], b[l], final=(l == L - 1))
    return h


# speedup vs baseline: 5.5490x; 1.0993x over previous
import functools

import jax
import jax.numpy as jnp
from jax import lax
from jax.experimental import pallas as pl
from jax.experimental.pallas import tpu as pltpu
from jax.experimental.pallas import tpu_sc as plsc

N = 50000
E = 400000
D = 128
L = 3
ET = 2

NS = 16
CW = 32
NK = D // CW
B = 200
EPT = E // NS
NB = EPT // B
CPB = 5
NCH = NB // CPB
NBR = 2 * E // B
NP = 50176
RPT = NP // NS
DR = 448
ND = RPT // DR

_MESH = dict(
    mesh=plsc.VectorSubcoreMesh(core_axis_name="c", subcore_axis_name="s"),
    compiler_params=pltpu.CompilerParams(use_tc_tiling_on_sc=False),
)


def _make_msum(with_deg):
    out_type = [jax.ShapeDtypeStruct((ET, NP, D), jnp.float32)]
    if with_deg:
        out_type = [jax.ShapeDtypeStruct((ET, NP, D), jnp.float32)] + out_type

    def body(hv, epk, zz, oo, *rest):
        if with_deg:
            dg, mo = rest[0], rest[1]
            scr = rest[2:]
        else:
            mo = rest[0]
            scr = rest[1:]
        acc, srcb, dstb, rows4, gsem, ssem, isem, zsem = scr
        c = lax.axis_index("c")
        s = lax.axis_index("s")
        slab0 = s * RPT
        gb0 = c * (E // B) + s * NB

        def zero_acc():
            for j in range(ND):
                pltpu.async_copy(zz, acc.at[pl.ds(slab0 + j * DR, DR)], zsem)
            for j in range(ND):
                pltpu.make_async_copy(zz, acc.at[pl.ds(slab0 + j * DR, DR)], zsem).wait()

        def drain(out_ref, k):
            col = pl.ds(k * CW, CW)
            for j in range(ND):
                r0 = slab0 + j * DR
                pltpu.async_copy(acc.at[pl.ds(r0, DR)],
                                 out_ref.at[c, pl.ds(r0, DR), col], zsem)
            for j in range(ND):
                r0 = slab0 + j * DR
                pltpu.make_async_copy(acc.at[pl.ds(r0, DR)],
                                      out_ref.at[c, pl.ds(r0, DR), col], zsem).wait()

        def edge_loop(k):
            gather = k is not None
            nidx = 2 if gather else 1

            def chunk_load_async(ch, slot):
                row = pl.ds(gb0 + ch * CPB, CPB)
                if gather:
                    pltpu.async_copy(epk.at[k, row], srcb.at[slot], isem)
                pltpu.async_copy(epk.at[NK, row], dstb.at[slot], isem)

            def chunk_load_wait():
                for _ in range(nidx):
                    pltpu.make_async_copy(
                        epk.at[NK, pl.ds(gb0, CPB)], dstb.at[0], isem).wait()

            def body_fn(j, carry):
                p = j % 4
                sp = j % 2
                q = (j // CPB) % 2
                r = j % CPB
                if gather:
                    pltpu.make_async_copy(
                        hv.at[srcb.at[q, r]], rows4.at[p], gsem.at[sp]).wait()
                src_slot = rows4.at[p] if gather else rows4.at[0]

                @pl.when(j >= 2)
                def _():
                    pltpu.make_async_copy(
                        rows4.at[(j + 2) % 4] if gather else rows4.at[0],
                        acc.at[dstb.at[q, r]], ssem.at[sp]).wait()

                ch1 = j // CPB + 1
                @pl.when((r == 2) & (ch1 < NCH))
                def _():
                    chunk_load_async(ch1, 1 - q)

                if gather:
                    @pl.when(j + 2 < NB)
                    def _():
                        j2 = j + 2
                        q2 = (j2 // CPB) % 2
                        r2 = j2 % CPB
                        @pl.when(r2 == 0)
                        def _():
                            chunk_load_wait()
                        pltpu.async_copy(
                            hv.at[srcb.at[q2, r2]], rows4.at[j2 % 4], gsem.at[sp])
                else:
                    @pl.when((r == CPB - 1) & (j + 1 < NB))
                    def _():
                        chunk_load_wait()

                pltpu.async_copy(src_slot, acc.at[dstb.at[q, r]], ssem.at[sp],
                                 add=True)
                return carry

            chunk_load_async(0, 0)
            chunk_load_wait()
            if gather:
                pltpu.async_copy(hv.at[srcb.at[0, 0]], rows4.at[0], gsem.at[0])
                pltpu.async_copy(hv.at[srcb.at[0, 1]], rows4.at[1], gsem.at[1])
            lax.fori_loop(0, NB, body_fn, 0)
            pltpu.make_async_copy(
                rows4.at[(NB - 2) % 4] if gather else rows4.at[0],
                acc.at[dstb.at[0, 0]], ssem.at[(NB - 2) % 2]).wait()
            pltpu.make_async_copy(
                rows4.at[(NB - 1) % 4] if gather else rows4.at[0],
                acc.at[dstb.at[0, 0]], ssem.at[(NB - 1) % 2]).wait()

        if with_deg:
            zero_acc()
            pltpu.sync_copy(oo, rows4.at[0])
            plsc.subcore_barrier()
            edge_loop(None)
            plsc.subcore_barrier()
            drain(dg, 0)
            plsc.subcore_barrier()
        for k in range(NK):
            zero_acc()
            plsc.subcore_barrier()
            edge_loop(k)
            plsc.subcore_barrier()
            drain(mo, k)
            plsc.subcore_barrier()

    return pl.kernel(
        body,
        out_type=out_type,
        scratch_types=[
            pltpu.VMEM_SHARED((NP, CW), jnp.float32),
            pltpu.VMEM((2, CPB, B), jnp.int32),
            pltpu.VMEM((2, CPB, B), jnp.int32),
            pltpu.VMEM((4, B, CW), jnp.float32),
            pltpu.SemaphoreType.DMA((2,)),
            pltpu.SemaphoreType.DMA((2,)),
            pltpu.SemaphoreType.DMA,
            pltpu.SemaphoreType.DMA,
        ],
        **_MESH,
    )


_msum0 = _make_msum(with_deg=True)
_msum = _make_msum(with_deg=False)


BNS = 1568
BNF = 1000


def _self_body(h_ref, ws_ref, b_ref, out_ref):
    h = h_ref[...]
    for e in range(ET):
        out_ref[e] = (
            jnp.dot(h, ws_ref[e], preferred_element_type=jnp.float32) + b_ref[e]
        )


def _dense_self(h, ws, bb):
    return pl.pallas_call(
        _self_body,
        grid=(NP // BNS,),
        in_specs=[
            pl.BlockSpec((BNS, D), lambda i: (i, 0)),
            pl.BlockSpec((ET, D, D), lambda i: (0, 0, 0)),
            pl.BlockSpec((ET, D), lambda i: (0, 0)),
        ],
        out_specs=pl.BlockSpec((ET, BNS, D), lambda i: (0, i, 0)),
        out_shape=jax.ShapeDtypeStruct((ET, NP, D), jnp.float32),
    )(h, ws, bb)


def _combine_body(s_ref, m_ref, deg_ref, wn_ref, out_ref, *, final, bn):
    out = jnp.zeros((bn, D), jnp.float32)
    for e in range(ET):
        inv = 1.0 / jnp.maximum(deg_ref[e][:, 0:1], 1.0)
        he = s_ref[e] + jnp.dot(m_ref[e] * inv, wn_ref[e],
                                preferred_element_type=jnp.float32)
        if not final:
            he = jnp.maximum(he, 0.0)
        out = out + he
    out_ref[...] = out


def _dense_combine(s, ms, deg, wn, *, final):
    bn = BNF if final else BNS
    rows = N if final else NP
    return pl.pallas_call(
        functools.partial(_combine_body, final=final, bn=bn),
        grid=(rows // bn,),
        in_specs=[
            pl.BlockSpec((ET, bn, D), lambda i: (0, i, 0)),
            pl.BlockSpec((ET, bn, D), lambda i: (0, i, 0)),
            pl.BlockSpec((ET, bn, D), lambda i: (0, i, 0)),
            pl.BlockSpec((ET, D, D), lambda i: (0, 0, 0)),
        ],
        out_specs=pl.BlockSpec((bn, D), lambda i: (i, 0)),
        out_shape=jax.ShapeDtypeStruct((rows, D), jnp.float32),
    )(s, ms, deg, wn)


def kernel(x, edge_index_0, edge_index_1, W_self, W_neigh, b):
    se = jnp.concatenate([edge_index_0[0], edge_index_1[0]]).reshape(NBR, B)
    de = jnp.concatenate([edge_index_0[1], edge_index_1[1]]).reshape(NBR, B)
    epk = jnp.stack([se * NK + k for k in range(NK)] + [de])
    zz = jnp.zeros((DR, CW), jnp.float32)
    oo = jnp.ones((B, CW), jnp.float32)
    h = jnp.pad(x, ((0, NP - N), (0, 0)))
    deg = None
    for l in range(L):
        final = l == L - 1
        hv = h.reshape(NP * NK, CW)
        if l == 0:
            deg, ms = _msum0(hv, epk, zz, oo)
        else:
            (ms,) = _msum(hv, epk, zz, oo)
        s = _dense_self(h, W_self[l], b[l])
        h = _dense_combine(s, ms, deg, W_neigh[l], final=final)
    return h


# speedup vs baseline: 5.6633x; 1.0206x over previous
import functools

import jax
import jax.numpy as jnp
from jax import lax
from jax.experimental import pallas as pl
from jax.experimental.pallas import tpu as pltpu
from jax.experimental.pallas import tpu_sc as plsc

N = 50000
E = 400000
D = 128
L = 3
ET = 2

NS = 16
CW = 32
NK = D // CW
B = 200
EPT = E // NS
NB = EPT // B
CPB = 5
NCH = NB // CPB
NBR = 2 * E // B
NP = 50176
RPT = NP // NS
DR = 448
ND = RPT // DR

_MESH = dict(
    mesh=plsc.VectorSubcoreMesh(core_axis_name="c", subcore_axis_name="s"),
    compiler_params=pltpu.CompilerParams(use_tc_tiling_on_sc=False),
)


def _make_msum(with_deg):
    out_type = [jax.ShapeDtypeStruct((ET, NP, D), jnp.float32)]
    if with_deg:
        out_type = [jax.ShapeDtypeStruct((ET, NP, D), jnp.float32)] + out_type

    def body(hv, epk, zz, oo, *rest):
        if with_deg:
            dg, mo = rest[0], rest[1]
            scr = rest[2:]
        else:
            mo = rest[0]
            scr = rest[1:]
        acc, srcb, dstb, rows4, gsem, ssem, isem, zsem = scr
        c = lax.axis_index("c")
        s = lax.axis_index("s")
        slab0 = s * RPT
        gb0 = c * (E // B) + s * NB

        def zero_acc():
            for j in range(ND):
                pltpu.async_copy(zz, acc.at[pl.ds(slab0 + j * DR, DR)], zsem)
            for j in range(ND):
                pltpu.make_async_copy(zz, acc.at[pl.ds(slab0 + j * DR, DR)], zsem).wait()

        def drain(out_ref, k):
            col = pl.ds(k * CW, CW)
            for j in range(ND):
                r0 = slab0 + j * DR
                pltpu.async_copy(acc.at[pl.ds(r0, DR)],
                                 out_ref.at[c, pl.ds(r0, DR), col], zsem)
            for j in range(ND):
                r0 = slab0 + j * DR
                pltpu.make_async_copy(acc.at[pl.ds(r0, DR)],
                                      out_ref.at[c, pl.ds(r0, DR), col], zsem).wait()

        def edge_loop(k):
            gather = k is not None
            nidx = 2 if gather else 1

            tbl = hv.at[pl.ds(k if gather else 0, NK * (NP - 1) + 1)]

            def chunk_load_async(ch, slot):
                row = pl.ds(gb0 + ch * CPB, CPB)
                if gather:
                    pltpu.async_copy(epk.at[0, row], srcb.at[slot], isem)
                pltpu.async_copy(epk.at[1, row], dstb.at[slot], isem)

            def chunk_load_wait():
                for _ in range(nidx):
                    pltpu.make_async_copy(
                        epk.at[1, pl.ds(gb0, CPB)], dstb.at[0], isem).wait()

            def body_fn(j, carry):
                p = j % 4
                sp = j % 2
                q = (j // CPB) % 2
                r = j % CPB
                if gather:
                    pltpu.make_async_copy(
                        tbl.at[srcb.at[q, r]], rows4.at[p], gsem.at[sp]).wait()
                src_slot = rows4.at[p] if gather else rows4.at[0]

                @pl.when(j >= 2)
                def _():
                    pltpu.make_async_copy(
                        rows4.at[(j + 2) % 4] if gather else rows4.at[0],
                        acc.at[dstb.at[q, r]], ssem.at[sp]).wait()

                ch1 = j // CPB + 1
                @pl.when((r == 2) & (ch1 < NCH))
                def _():
                    chunk_load_async(ch1, 1 - q)

                if gather:
                    @pl.when(j + 2 < NB)
                    def _():
                        j2 = j + 2
                        q2 = (j2 // CPB) % 2
                        r2 = j2 % CPB
                        @pl.when(r2 == 0)
                        def _():
                            chunk_load_wait()
                        pltpu.async_copy(
                            tbl.at[srcb.at[q2, r2]], rows4.at[j2 % 4], gsem.at[sp])
                else:
                    @pl.when((r == CPB - 1) & (j + 1 < NB))
                    def _():
                        chunk_load_wait()

                pltpu.async_copy(src_slot, acc.at[dstb.at[q, r]], ssem.at[sp],
                                 add=True)
                return carry

            chunk_load_async(0, 0)
            chunk_load_wait()
            if gather:
                pltpu.async_copy(tbl.at[srcb.at[0, 0]], rows4.at[0], gsem.at[0])
                pltpu.async_copy(tbl.at[srcb.at[0, 1]], rows4.at[1], gsem.at[1])
            lax.fori_loop(0, NB, body_fn, 0)
            pltpu.make_async_copy(
                rows4.at[(NB - 2) % 4] if gather else rows4.at[0],
                acc.at[dstb.at[0, 0]], ssem.at[(NB - 2) % 2]).wait()
            pltpu.make_async_copy(
                rows4.at[(NB - 1) % 4] if gather else rows4.at[0],
                acc.at[dstb.at[0, 0]], ssem.at[(NB - 1) % 2]).wait()

        if with_deg:
            zero_acc()
            pltpu.sync_copy(oo, rows4.at[0])
            plsc.subcore_barrier()
            edge_loop(None)
            plsc.subcore_barrier()
            drain(dg, 0)
            plsc.subcore_barrier()
        for k in range(NK):
            zero_acc()
            plsc.subcore_barrier()
            edge_loop(k)
            plsc.subcore_barrier()
            drain(mo, k)
            plsc.subcore_barrier()

    return pl.kernel(
        body,
        out_type=out_type,
        scratch_types=[
            pltpu.VMEM_SHARED((NP, CW), jnp.float32),
            pltpu.VMEM((2, CPB, B), jnp.int32),
            pltpu.VMEM((2, CPB, B), jnp.int32),
            pltpu.VMEM((4, B, CW), jnp.float32),
            pltpu.SemaphoreType.DMA((2,)),
            pltpu.SemaphoreType.DMA((2,)),
            pltpu.SemaphoreType.DMA,
            pltpu.SemaphoreType.DMA,
        ],
        **_MESH,
    )


_msum0 = _make_msum(with_deg=True)
_msum = _make_msum(with_deg=False)


BNS = 1568
BNF = 1000


def _self_body(h_ref, ws_ref, b_ref, out_ref):
    h = h_ref[...]
    for e in range(ET):
        out_ref[e] = (
            jnp.dot(h, ws_ref[e], preferred_element_type=jnp.float32) + b_ref[e]
        )


def _dense_self(h, ws, bb):
    return pl.pallas_call(
        _self_body,
        grid=(NP // BNS,),
        in_specs=[
            pl.BlockSpec((BNS, D), lambda i: (i, 0)),
            pl.BlockSpec((ET, D, D), lambda i: (0, 0, 0)),
            pl.BlockSpec((ET, D), lambda i: (0, 0)),
        ],
        out_specs=pl.BlockSpec((ET, BNS, D), lambda i: (0, i, 0)),
        out_shape=jax.ShapeDtypeStruct((ET, NP, D), jnp.float32),
    )(h, ws, bb)


def _combine_body(s_ref, m_ref, deg_ref, wn_ref, out_ref, *, final, bn):
    out = jnp.zeros((bn, D), jnp.float32)
    for e in range(ET):
        inv = 1.0 / jnp.maximum(deg_ref[e], 1.0)
        he = s_ref[e] + jnp.dot(m_ref[e] * inv, wn_ref[e],
                                preferred_element_type=jnp.float32)
        if not final:
            he = jnp.maximum(he, 0.0)
        out = out + he
    out_ref[...] = out


def _dense_combine(s, ms, deg, wn, *, final):
    bn = BNF if final else BNS
    rows = N if final else NP
    return pl.pallas_call(
        functools.partial(_combine_body, final=final, bn=bn),
        grid=(rows // bn,),
        in_specs=[
            pl.BlockSpec((ET, bn, D), lambda i: (0, i, 0)),
            pl.BlockSpec((ET, bn, D), lambda i: (0, i, 0)),
            pl.BlockSpec((ET, bn, 1), lambda i: (0, i, 0)),
            pl.BlockSpec((ET, D, D), lambda i: (0, 0, 0)),
        ],
        out_specs=pl.BlockSpec((bn, D), lambda i: (i, 0)),
        out_shape=jax.ShapeDtypeStruct((rows, D), jnp.float32),
    )(s, ms, deg, wn)


def kernel(x, edge_index_0, edge_index_1, W_self, W_neigh, b):
    se = jnp.concatenate([edge_index_0[0], edge_index_1[0]]).reshape(NBR, B)
    de = jnp.concatenate([edge_index_0[1], edge_index_1[1]]).reshape(NBR, B)
    epk = jnp.stack([se * NK, de])
    zz = jnp.zeros((DR, CW), jnp.float32)
    oo = jnp.ones((B, CW), jnp.float32)
    h = jnp.pad(x, ((0, NP - N), (0, 0)))
    deg = None
    for l in range(L):
        final = l == L - 1
        hv = h.reshape(NP * NK, CW)
        if l == 0:
            dg, ms = _msum0(hv, epk, zz, oo)
            deg = dg[:, :, 0:1]
        else:
            (ms,) = _msum(hv, epk, zz, oo)
        s = _dense_self(h, W_self[l], b[l])
        h = _dense_combine(s, ms, deg, W_neigh[l], final=final)
    return h


# speedup vs baseline: 6.0319x; 1.0651x over previous
import functools

import jax
import jax.numpy as jnp
from jax import lax
from jax.experimental import pallas as pl
from jax.experimental.pallas import tpu as pltpu
from jax.experimental.pallas import tpu_sc as plsc

N = 50000
E = 400000
D = 128
L = 3
ET = 2

NS = 16
CW = 32
NK = D // CW
B = 200
EPT = E // NS
NB = EPT // B
CPB = 5
NCH = NB // CPB
NBR = 2 * E // B
NP = 50176
RPT = NP // NS
DR = 448
ND = RPT // DR

_MESH = dict(
    mesh=plsc.VectorSubcoreMesh(core_axis_name="c", subcore_axis_name="s"),
    compiler_params=pltpu.CompilerParams(use_tc_tiling_on_sc=False),
)


def _make_msum(with_deg):
    out_type = [jax.ShapeDtypeStruct((ET, NP, D), jnp.float32)]
    if with_deg:
        out_type = [jax.ShapeDtypeStruct((ET, NP, D), jnp.float32)] + out_type

    def body(hv, epk, zz, oo, *rest):
        if with_deg:
            dg, mo = rest[0], rest[1]
            scr = rest[2:]
        else:
            mo = rest[0]
            scr = rest[1:]
        acc, srcb, dstb, rows4, gsem, ssem, isem, zsem, z2sem = scr
        c = lax.axis_index("c")
        s = lax.axis_index("s")
        slab0 = s * RPT
        gb0 = c * (E // B) + s * NB

        def zero_acc():
            for j in range(ND):
                pltpu.async_copy(zz, acc.at[pl.ds(slab0 + j * DR, DR)], zsem)
            for j in range(ND):
                pltpu.make_async_copy(zz, acc.at[pl.ds(slab0 + j * DR, DR)], zsem).wait()

        def drain_and_zero(out_ref, k, do_zero):
            col = pl.ds(k * CW, CW)
            for j in range(ND):
                r0 = slab0 + j * DR
                pltpu.async_copy(acc.at[pl.ds(r0, DR)],
                                 out_ref.at[c, pl.ds(r0, DR), col], zsem)
            for j in range(ND):
                r0 = slab0 + j * DR
                pltpu.make_async_copy(acc.at[pl.ds(r0, DR)],
                                      out_ref.at[c, pl.ds(r0, DR), col], zsem).wait()
                if do_zero:
                    pltpu.async_copy(zz, acc.at[pl.ds(r0, DR)], z2sem)
            if do_zero:
                for j in range(ND):
                    pltpu.make_async_copy(
                        zz, acc.at[pl.ds(slab0 + j * DR, DR)], z2sem).wait()

        def edge_loop(k):
            gather = k is not None
            nidx = 2 if gather else 1

            tbl = hv.at[pl.ds(k if gather else 0, NK * (NP - 1) + 1)]

            def chunk_load_async(ch, slot):
                row = pl.ds(gb0 + ch * CPB, CPB)
                if gather:
                    pltpu.async_copy(epk.at[0, row], srcb.at[slot], isem)
                pltpu.async_copy(epk.at[1, row], dstb.at[slot], isem)

            def chunk_load_wait():
                for _ in range(nidx):
                    pltpu.make_async_copy(
                        epk.at[1, pl.ds(gb0, CPB)], dstb.at[0], isem).wait()

            def body_fn(j, carry):
                p = j % 4
                sp = j % 2
                q = (j // CPB) % 2
                r = j % CPB
                if gather:
                    pltpu.make_async_copy(
                        tbl.at[srcb.at[q, r]], rows4.at[p], gsem.at[sp]).wait()
                src_slot = rows4.at[p] if gather else rows4.at[0]

                @pl.when(j >= 2)
                def _():
                    pltpu.make_async_copy(
                        rows4.at[(j + 2) % 4] if gather else rows4.at[0],
                        acc.at[dstb.at[q, r]], ssem.at[sp]).wait()

                ch1 = j // CPB + 1
                @pl.when((r == 2) & (ch1 < NCH))
                def _():
                    chunk_load_async(ch1, 1 - q)

                if gather:
                    @pl.when(j + 2 < NB)
                    def _():
                        j2 = j + 2
                        q2 = (j2 // CPB) % 2
                        r2 = j2 % CPB
                        @pl.when(r2 == 0)
                        def _():
                            chunk_load_wait()
                        pltpu.async_copy(
                            tbl.at[srcb.at[q2, r2]], rows4.at[j2 % 4], gsem.at[sp])
                else:
                    @pl.when((r == CPB - 1) & (j + 1 < NB))
                    def _():
                        chunk_load_wait()

                pltpu.async_copy(src_slot, acc.at[dstb.at[q, r]], ssem.at[sp],
                                 add=True)
                return carry

            chunk_load_async(0, 0)
            chunk_load_wait()
            if gather:
                pltpu.async_copy(tbl.at[srcb.at[0, 0]], rows4.at[0], gsem.at[0])
                pltpu.async_copy(tbl.at[srcb.at[0, 1]], rows4.at[1], gsem.at[1])
            lax.fori_loop(0, NB, body_fn, 0)
            pltpu.make_async_copy(
                rows4.at[(NB - 2) % 4] if gather else rows4.at[0],
                acc.at[dstb.at[0, 0]], ssem.at[(NB - 2) % 2]).wait()
            pltpu.make_async_copy(
                rows4.at[(NB - 1) % 4] if gather else rows4.at[0],
                acc.at[dstb.at[0, 0]], ssem.at[(NB - 1) % 2]).wait()

        zero_acc()
        if with_deg:
            pltpu.sync_copy(oo, rows4.at[0])
        plsc.subcore_barrier()
        if with_deg:
            edge_loop(None)
            plsc.subcore_barrier()
            drain_and_zero(dg, 0, True)
            plsc.subcore_barrier()
        for k in range(NK):
            edge_loop(k)
            plsc.subcore_barrier()
            drain_and_zero(mo, k, k < NK - 1)
            if k < NK - 1:
                plsc.subcore_barrier()

    return pl.kernel(
        body,
        out_type=out_type,
        scratch_types=[
            pltpu.VMEM_SHARED((NP, CW), jnp.float32),
            pltpu.VMEM((2, CPB, B), jnp.int32),
            pltpu.VMEM((2, CPB, B), jnp.int32),
            pltpu.VMEM((4, B, CW), jnp.float32),
            pltpu.SemaphoreType.DMA((2,)),
            pltpu.SemaphoreType.DMA((2,)),
            pltpu.SemaphoreType.DMA,
            pltpu.SemaphoreType.DMA,
            pltpu.SemaphoreType.DMA,
        ],
        **_MESH,
    )


_msum0 = _make_msum(with_deg=True)
_msum = _make_msum(with_deg=False)


BNS = 1568
BNF = 1000


def _self_body(h_ref, ws_ref, b_ref, out_ref):
    h = h_ref[...]
    for e in range(ET):
        out_ref[e] = (
            jnp.dot(h, ws_ref[e], preferred_element_type=jnp.float32) + b_ref[e]
        )


def _dense_self(h, ws, bb):
    return pl.pallas_call(
        _self_body,
        grid=(NP // BNS,),
        in_specs=[
            pl.BlockSpec((BNS, D), lambda i: (i, 0)),
            pl.BlockSpec((ET, D, D), lambda i: (0, 0, 0)),
            pl.BlockSpec((ET, D), lambda i: (0, 0)),
        ],
        out_specs=pl.BlockSpec((ET, BNS, D), lambda i: (0, i, 0)),
        out_shape=jax.ShapeDtypeStruct((ET, NP, D), jnp.float32),
    )(h, ws, bb)


def _combine_body(s_ref, m_ref, deg_ref, wn_ref, out_ref, *, final, bn):
    out = jnp.zeros((bn, D), jnp.float32)
    for e in range(ET):
        inv = 1.0 / jnp.maximum(deg_ref[e], 1.0)
        he = s_ref[e] + jnp.dot(m_ref[e] * inv, wn_ref[e],
                                preferred_element_type=jnp.float32)
        if not final:
            he = jnp.maximum(he, 0.0)
        out = out + he
    out_ref[...] = out


def _dense_combine(s, ms, deg, wn, *, final):
    bn = BNF if final else BNS
    rows = N if final else NP
    return pl.pallas_call(
        functools.partial(_combine_body, final=final, bn=bn),
        grid=(rows // bn,),
        in_specs=[
            pl.BlockSpec((ET, bn, D), lambda i: (0, i, 0)),
            pl.BlockSpec((ET, bn, D), lambda i: (0, i, 0)),
            pl.BlockSpec((ET, bn, 1), lambda i: (0, i, 0)),
            pl.BlockSpec((ET, D, D), lambda i: (0, 0, 0)),
        ],
        out_specs=pl.BlockSpec((bn, D), lambda i: (i, 0)),
        out_shape=jax.ShapeDtypeStruct((rows, D), jnp.float32),
    )(s, ms, deg, wn)


def kernel(x, edge_index_0, edge_index_1, W_self, W_neigh, b):
    se = jnp.concatenate([edge_index_0[0], edge_index_1[0]]).reshape(NBR, B)
    de = jnp.concatenate([edge_index_0[1], edge_index_1[1]]).reshape(NBR, B)
    epk = jnp.stack([se * NK, de])
    zz = jnp.zeros((DR, CW), jnp.float32)
    oo = jnp.ones((B, CW), jnp.float32)
    h = jnp.pad(x, ((0, NP - N), (0, 0)))
    deg = None
    for l in range(L):
        final = l == L - 1
        hv = h.reshape(NP * NK, CW)
        if l == 0:
            dg, ms = _msum0(hv, epk, zz, oo)
            deg = dg[:, :, 0:1]
        else:
            (ms,) = _msum(hv, epk, zz, oo)
        s = _dense_self(h, W_self[l], b[l])
        h = _dense_combine(s, ms, deg, W_neigh[l], final=final)
    return h
